# baseline (speedup 1.0000x reference)
import functools

import jax
import jax.numpy as jnp
from jax import lax
from jax.experimental import pallas as pl
from jax.experimental.pallas import tpu as pltpu
from jax.experimental.pallas import tpu_sc as plsc

_NUM_CORES = 2
_NUM_SUBCORES = 16
_NUM_WORKERS = _NUM_CORES * _NUM_SUBCORES
_CHUNK = 128


def kernel(x, table):
    (batch,) = x.shape
    vocab, dim = table.shape
    b_per_w = batch // _NUM_WORKERS
    n_chunks = b_per_w // _CHUNK
    wide = 2 * dim

    idx = x.astype(jnp.int32)
    t2 = table.reshape(vocab // 2, wide)
    mesh = plsc.VectorSubcoreMesh(
        core_axis_name="c", subcore_axis_name="s",
        num_cores=_NUM_CORES, num_subcores=_NUM_SUBCORES)

    @functools.partial(
        pl.kernel,
        out_type=jax.ShapeDtypeStruct((batch, dim), table.dtype),
        mesh=mesh,
        scratch_types=[
            pltpu.VMEM((b_per_w,), jnp.int32),
            pltpu.VMEM((b_per_w,), jnp.int32),
            pltpu.VMEM((b_per_w // 2, wide), jnp.float32),
            pltpu.VMEM((b_per_w, dim), jnp.float32),
            pltpu.SemaphoreType.DMA,
        ],
        compiler_params=pltpu.CompilerParams(needs_layout_passes=False),
    )
    def emb(idx_hbm, t2_hbm, out_hbm, idx_v, widx_v, wide_v, rows_v, gsem):
        wid = lax.axis_index("s") * _NUM_CORES + lax.axis_index("c")
        base = wid * b_per_w
        pltpu.sync_copy(idx_hbm.at[pl.ds(base, b_per_w)], idx_v)

        def widx_body(j, carry):
            v = idx_v[pl.ds(j * 16, 16)]
            widx_v[pl.ds(j * 16, 16)] = lax.shift_right_logical(v, 1)
            return carry

        lax.fori_loop(0, b_per_w // 16, widx_body, 0)

        half = b_per_w // 2
        for hblk in range(2):
            for j in range(half // _CHUNK):
                c = hblk * half + j * _CHUNK
                pltpu.make_async_copy(
                    t2_hbm.at[widx_v.at[pl.ds(c, _CHUNK)]],
                    wide_v.at[pl.ds(j * _CHUNK, _CHUNK), :],
                    gsem).start()
            pltpu.make_async_copy(
                t2_hbm.at[pl.ds(0, half)], wide_v, gsem).wait()

            def sel_body(j, carry, hblk=hblk):
                v = idx_v[pl.ds(hblk * half + j * 16, 16)]
                hoff = (v & 1) * dim
                rows16 = j * 16 + lax.iota(jnp.int32, 16)
                orows16 = hblk * half + rows16
                for d in range(dim):
                    vals = plsc.load_gather(wide_v, [rows16, hoff + d])
                    plsc.store_scatter(
                        rows_v,
                        [orows16, jnp.full((16,), d, jnp.int32)], vals)
                return carry

            lax.fori_loop(0, half // 16, sel_body, 0)
        pltpu.sync_copy(rows_v, out_hbm.at[pl.ds(base, b_per_w), :])

    return emb(idx, t2)


# speedup vs baseline: 1.3420x; 1.3420x over previous
import functools

import jax
import jax.numpy as jnp
from jax import lax
from jax.experimental import pallas as pl
from jax.experimental.pallas import tpu as pltpu
from jax.experimental.pallas import tpu_sc as plsc

_NUM_CORES = 2
_NUM_SUBCORES = 16
_NUM_WORKERS = _NUM_CORES * _NUM_SUBCORES
_CHUNK = 128


def kernel(x, table):
    (batch,) = x.shape
    _, dim = table.shape
    b_per_w = batch // _NUM_WORKERS
    n_chunks = b_per_w // _CHUNK

    idx = x.astype(jnp.int32)
    mesh = plsc.VectorSubcoreMesh(
        core_axis_name="c", subcore_axis_name="s",
        num_cores=_NUM_CORES, num_subcores=_NUM_SUBCORES)

    @functools.partial(
        pl.kernel,
        out_type=jax.ShapeDtypeStruct((batch, dim), table.dtype),
        mesh=mesh,
        scratch_types=[
            pltpu.VMEM((b_per_w,), jnp.int32),
            pltpu.VMEM((b_per_w, dim), jnp.float32),
            pltpu.SemaphoreType.DMA,
        ],
        compiler_params=pltpu.CompilerParams(use_tc_tiling_on_sc=False),
    )
    def emb(idx_hbm, table_hbm, out_hbm, idx_v, rows_v, sem):
        wid = lax.axis_index("s") * _NUM_CORES + lax.axis_index("c")
        base = wid * b_per_w
        pltpu.sync_copy(idx_hbm.at[pl.ds(base, b_per_w)], idx_v)
        for j in range(n_chunks):
            pltpu.make_async_copy(
                table_hbm.at[idx_v.at[pl.ds(j * _CHUNK, _CHUNK)]],
                rows_v.at[pl.ds(j * _CHUNK, _CHUNK), :],
                sem).start()
        pltpu.make_async_copy(
            table_hbm.at[pl.ds(0, b_per_w)], rows_v, sem).wait()
        pltpu.sync_copy(rows_v, out_hbm.at[pl.ds(base, b_per_w), :])

    return emb(idx, table)


# speedup vs baseline: 2.0045x; 1.4936x over previous
import functools

import jax
import jax.numpy as jnp
from jax import lax
from jax.experimental import pallas as pl
from jax.experimental.pallas import tpu as pltpu
from jax.experimental.pallas import tpu_sc as plsc

_NUM_CORES = 2
_NUM_SUBCORES = 16
_NUM_WORKERS = _NUM_CORES * _NUM_SUBCORES


def kernel(x, table):
    (batch,) = x.shape
    _, dim = table.shape
    b_per_w = batch // _NUM_WORKERS

    idx = x.astype(jnp.int32)
    mesh = plsc.VectorSubcoreMesh(
        core_axis_name="c", subcore_axis_name="s",
        num_cores=_NUM_CORES, num_subcores=_NUM_SUBCORES)

    @functools.partial(
        pl.kernel,
        out_type=jax.ShapeDtypeStruct((batch, dim), table.dtype),
        mesh=mesh,
        scratch_types=[
            pltpu.VMEM((b_per_w,), jnp.int32),
            pltpu.VMEM((b_per_w, dim), jnp.float32),
            pltpu.SemaphoreType.DMA,
        ],
    )
    def emb(idx_hbm, table_hbm, out_hbm, idx_v, rows_v, sem):
        wid = lax.axis_index("s") * _NUM_CORES + lax.axis_index("c")
        base = wid * b_per_w
        pltpu.sync_copy(idx_hbm.at[pl.ds(base, b_per_w)], idx_v)

        def body(j, carry):
            for g in range(4):
                v = idx_v[pl.ds(j * 64 + g * 16, 16)]
                for k in range(16):
                    r = v[k]
                    pltpu.make_async_copy(
                        table_hbm.at[pl.ds(r, 1), :],
                        rows_v.at[pl.ds(j * 64 + g * 16 + k, 1), :],
                        sem).start()
            return carry

        lax.fori_loop(0, b_per_w // 64, body, 0)
        pltpu.make_async_copy(
            table_hbm.at[pl.ds(0, b_per_w), :], rows_v, sem).wait()
        pltpu.sync_copy(rows_v, out_hbm.at[pl.ds(base, b_per_w), :])

    return emb(idx, table)
